# baseline (speedup 1.0000x reference)
import functools

import jax
import jax.numpy as jnp
from jax import lax
from jax.experimental import pallas as pl
from jax.experimental.pallas import tpu as pltpu, tpu_sc as plsc

_info = plsc.get_sparse_core_info()
_NC, _NS, _L = _info.num_cores, _info.num_subcores, _info.num_lanes
_NCH = 8


@functools.lru_cache(maxsize=None)
def _build(B: int, S: int, D: int):
    NW = _NC * _NS
    bpw = B // NW
    ch = bpw // _NCH
    mesh = plsc.VectorSubcoreMesh(core_axis_name="c", subcore_axis_name="s")

    @functools.partial(
        pl.kernel,
        mesh=mesh,
        out_type=jax.ShapeDtypeStruct((B, D), jnp.float32),
        scratch_types=[
            pltpu.VMEM((16,), jnp.int32),
            pltpu.VMEM((bpw,), jnp.int32),
            pltpu.VMEM((bpw, D), jnp.float32),
            pltpu.SemaphoreType.DMA,
            pltpu.SemaphoreType.DMA,
        ],
    )
    def gather_rows(table_hbm, idx_hbm, out_hbm, idx_stage, row_idx, rows,
                    gsem, ssem):
        wid = lax.axis_index("s") * _NC + lax.axis_index("c")
        base = wid * bpw
        pltpu.sync_copy(idx_hbm, idx_stage)
        idx_vec = idx_stage[...]
        lane = lax.iota(jnp.int32, _L)
        gathers = []
        for c in range(_NCH):
            for j in range(c * ch // _L, (c + 1) * ch // _L):
                row_idx[pl.ds(j * _L, _L)] = (base + j * _L + lane) * S + idx_vec
            gathers.append(
                pltpu.async_copy(table_hbm.at[row_idx.at[pl.ds(c * ch, ch)]],
                                 rows.at[pl.ds(c * ch, ch)], gsem))
        scatters = []
        for c in range(_NCH):
            gathers[c].wait()
            scatters.append(
                pltpu.async_copy(rows.at[pl.ds(c * ch, ch)],
                                 out_hbm.at[pl.ds(base + c * ch, ch)], ssem))
        for s in scatters:
            s.wait()

    return gather_rows


def kernel(input, indices):
    B, S, D = input.shape
    table = input.reshape(B * S, D)
    idx = jnp.broadcast_to(indices.astype(jnp.int32), (16,))
    out = _build(B, S, D)(table, idx)
    return out.reshape(B, 1, D)


# speedup vs baseline: 1.0280x; 1.0280x over previous
import functools

import jax
import jax.numpy as jnp
from jax import lax
from jax.experimental import pallas as pl
from jax.experimental.pallas import tpu as pltpu, tpu_sc as plsc

_info = plsc.get_sparse_core_info()
_NC, _NS, _L = _info.num_cores, _info.num_subcores, _info.num_lanes
_NCH = 4


@functools.lru_cache(maxsize=None)
def _build(B: int, S: int, D: int):
    NW = _NC * _NS
    bpw = B // NW
    ch = bpw // _NCH
    mesh = plsc.VectorSubcoreMesh(core_axis_name="c", subcore_axis_name="s")

    @functools.partial(
        pl.kernel,
        mesh=mesh,
        out_type=jax.ShapeDtypeStruct((B, D), jnp.float32),
        scratch_types=[
            pltpu.VMEM((16,), jnp.int32),
            pltpu.VMEM((bpw,), jnp.int32),
            pltpu.VMEM((bpw, D), jnp.float32),
            pltpu.SemaphoreType.DMA,
            pltpu.SemaphoreType.DMA,
        ],
    )
    def gather_rows(table_hbm, idx_hbm, out_hbm, idx_stage, row_idx, rows,
                    gsem, ssem):
        wid = lax.axis_index("s") * _NC + lax.axis_index("c")
        base = wid * bpw
        pltpu.sync_copy(idx_hbm, idx_stage)
        idx_vec = idx_stage[...]
        lane = lax.iota(jnp.int32, _L)
        gathers = []
        for c in range(_NCH):
            for j in range(c * ch // _L, (c + 1) * ch // _L):
                row_idx[pl.ds(j * _L, _L)] = (base + j * _L + lane) * S + idx_vec
            gathers.append(
                pltpu.async_copy(table_hbm.at[row_idx.at[pl.ds(c * ch, ch)]],
                                 rows.at[pl.ds(c * ch, ch)], gsem))
        scatters = []
        for c in range(_NCH):
            gathers[c].wait()
            scatters.append(
                pltpu.async_copy(rows.at[pl.ds(c * ch, ch)],
                                 out_hbm.at[pl.ds(base + c * ch, ch)], ssem))
        for s in scatters:
            s.wait()

    return gather_rows


def kernel(input, indices):
    B, S, D = input.shape
    table = input.reshape(B * S, D)
    idx = jnp.broadcast_to(indices.astype(jnp.int32), (16,))
    out = _build(B, S, D)(table, idx)
    return out.reshape(B, 1, D)
